# baseline (speedup 1.0000x reference)
import functools

import jax
import jax.numpy as jnp
from jax import lax
from jax.experimental import pallas as pl
from jax.experimental.pallas import tpu as pltpu
from jax.experimental.pallas import tpu_sc as plsc

BLK = 256


def _sc_dispatch(x, tflat, padpos3, n_out, chunk):
    _, H = x.shape
    NW, nchunks, _ = padpos3.shape
    info = plsc.get_sparse_core_info()
    NC, NS = info.num_cores, info.num_subcores
    assert NW == NC * NS and padpos3.shape[2] == chunk
    per_w = nchunks * chunk
    mesh = plsc.VectorSubcoreMesh(core_axis_name="c", subcore_axis_name="s")

    @functools.partial(
        pl.kernel,
        out_type=jax.ShapeDtypeStruct((n_out, H), x.dtype),
        mesh=mesh,
        scratch_types=[
            pltpu.VMEM((per_w,), jnp.int32),
            pltpu.VMEM((nchunks, chunk), jnp.int32),
            pltpu.VMEM((chunk, H), x.dtype),
            pltpu.VMEM((chunk, H), x.dtype),
            pltpu.SemaphoreType.DMA,
            pltpu.SemaphoreType.DMA,
        ],
    )
    def dispatch_k(x_hbm, tf_hbm, pp_hbm, out_hbm, idx_v, pp_v, b0, b1, gsem,
                   ssem):
        wid = lax.axis_index("s") * NC + lax.axis_index("c")
        base = wid * per_w
        pltpu.sync_copy(tf_hbm.at[pl.ds(base, per_w)], idx_v)
        pltpu.sync_copy(pp_hbm.at[wid], pp_v)
        bufs = [b0, b1]
        gathers = [None] * nchunks
        stores = [None] * nchunks
        gathers[0] = pltpu.async_copy(
            x_hbm.at[idx_v.at[pl.ds(0, chunk)]], bufs[0], gsem)
        for c in range(nchunks):
            buf = bufs[c % 2]
            gathers[c].wait()
            if c + 1 < nchunks:
                if c >= 1:
                    stores[c - 1].wait()
                gathers[c + 1] = pltpu.async_copy(
                    x_hbm.at[idx_v.at[pl.ds((c + 1) * chunk, chunk)]],
                    bufs[(c + 1) % 2], gsem)
            stores[c] = pltpu.async_copy(buf, out_hbm.at[pp_v.at[c]], ssem)
        for c in range(max(0, nchunks - 2), nchunks):
            stores[c].wait()

    return dispatch_k(x, tflat, padpos3)


def _sc_combine(ys, pos0, pos1, w0, w1):
    T = pos0.shape[0]
    H = ys.shape[1]
    info = plsc.get_sparse_core_info()
    NC, NS = info.num_cores, info.num_subcores
    NW = NC * NS
    per_w = T // NW
    assert T % NW == 0 and per_w % 8 == 0
    mesh = plsc.VectorSubcoreMesh(core_axis_name="c", subcore_axis_name="s")

    @functools.partial(
        pl.kernel,
        out_type=jax.ShapeDtypeStruct((T, H), ys.dtype),
        mesh=mesh,
        scratch_types=[
            pltpu.VMEM((per_w,), jnp.int32),
            pltpu.VMEM((per_w,), jnp.int32),
            pltpu.VMEM((per_w, 16), jnp.float32),
            pltpu.VMEM((per_w, 16), jnp.float32),
            pltpu.VMEM((per_w, H), ys.dtype),
            pltpu.VMEM((per_w, H), ys.dtype),
            pltpu.SemaphoreType.DMA,
        ],
    )
    def combine_k(ys_hbm, p0_hbm, p1_hbm, w0_hbm, w1_hbm, out_hbm, i0, i1,
                  wv0, wv1, b0, b1, sem):
        wid = lax.axis_index("s") * NC + lax.axis_index("c")
        base = wid * per_w
        pltpu.sync_copy(p0_hbm.at[pl.ds(base, per_w)], i0)
        pltpu.sync_copy(p1_hbm.at[pl.ds(base, per_w)], i1)
        pltpu.sync_copy(w0_hbm.at[pl.ds(base, per_w)], wv0)
        pltpu.sync_copy(w1_hbm.at[pl.ds(base, per_w)], wv1)
        cp0 = pltpu.async_copy(ys_hbm.at[i0], b0, sem)
        cp1 = pltpu.async_copy(ys_hbm.at[i1], b1, sem)
        cp0.wait()
        cp1.wait()

        def row(r, carry):
            a0 = wv0[r, :]
            a1 = wv1[r, :]
            for c in range(H // 16):
                s = pl.ds(c * 16, 16)
                b0[r, s] = b0[r, s] * a0 + b1[r, s] * a1
            return carry

        lax.fori_loop(0, per_w, row, 0)
        pltpu.sync_copy(b0, out_hbm.at[pl.ds(base, per_w)])

    return combine_k(ys, pos0, pos1, w0, w1)


def _ffn_body(I, E, ps_ref, adp_ref, scl_ref, xs_ref, wgu_ref, wd_ref,
              ga_ref, gb_ref, ua_ref, ub_ref, da_ref, db_ref, ys_ref):
    i = pl.program_id(0)

    @pl.when(i * BLK < ps_ref[E])
    def _():
        f32 = jnp.float32
        bf = jnp.bfloat16
        cT = (((1,), (1,)), ((), ()))
        s = scl_ref[adp_ref[0]]
        x = xs_ref[...].astype(bf)
        gb_base = jnp.dot(x, wgu_ref[0].astype(bf), preferred_element_type=f32)
        lg = lax.dot_general(
            lax.dot_general(x, ga_ref[0, 0].astype(bf), cT,
                            preferred_element_type=f32).astype(bf),
            gb_ref[0, 0].astype(bf), cT, preferred_element_type=f32)
        lu = lax.dot_general(
            lax.dot_general(x, ua_ref[0, 0].astype(bf), cT,
                            preferred_element_type=f32).astype(bf),
            ub_ref[0, 0].astype(bf), cT, preferred_element_type=f32)
        g = gb_base[:, :I] + s * lg
        u = gb_base[:, I:] + s * lu
        a = (g * jax.nn.sigmoid(g) * u).astype(bf)
        d = jnp.dot(a, wd_ref[0].astype(bf), preferred_element_type=f32)
        ld = lax.dot_general(
            lax.dot_general(a, da_ref[0, 0].astype(bf), cT,
                            preferred_element_type=f32).astype(bf),
            db_ref[0, 0].astype(bf), cT, preferred_element_type=f32)
        ys_ref[...] = d + s * ld


def kernel(hidden_states, topk_ids, topk_weights, gate_a, gate_b, up_a, up_b,
           down_a, down_b, weight_indices, seq_lens, lora_ranks, scalings,
           base_gate_up_weight, base_down_weight):
    T, H = hidden_states.shape
    K = topk_ids.shape[1]
    E = base_gate_up_weight.shape[0]
    I = base_down_weight.shape[1]
    R = gate_a.shape[2]
    NP = T * K
    NPAD = NP + E * BLK
    NBLK = NPAD // BLK
    NW = 32
    CHUNK = 64

    x = hidden_states.astype(jnp.float32)

    e_flat = topk_ids.reshape(-1).astype(jnp.int32)
    t_flat = jnp.repeat(jnp.arange(T, dtype=jnp.int32), K)
    oh = (e_flat[:, None] == jnp.arange(E, dtype=jnp.int32)[None, :])
    ranks_all = jnp.cumsum(oh.astype(jnp.int32), axis=0)
    counts = ranks_all[-1]
    rank = jnp.take_along_axis(ranks_all, e_flat[:, None], axis=1)[:, 0] - 1
    padded = ((counts + BLK - 1) // BLK) * BLK
    pad_start = jnp.concatenate(
        [jnp.zeros((1,), jnp.int32), jnp.cumsum(padded).astype(jnp.int32)])
    padpos = pad_start[e_flat] + rank
    pos = padpos.reshape(T, K)

    xs = _sc_dispatch(x, t_flat, padpos.reshape(NW, NP // (NW * CHUNK), CHUNK),
                      NPAD, CHUNK)

    def eid(i, ps, adp, scl):
        st = i * BLK
        e = jnp.int32(0)
        for j in range(1, E):
            e = e + (ps[j] <= st).astype(jnp.int32)
        return e

    grid_spec = pltpu.PrefetchScalarGridSpec(
        num_scalar_prefetch=3,
        grid=(NBLK,),
        in_specs=[
            pl.BlockSpec((BLK, H), lambda i, *sp: (i, 0)),
            pl.BlockSpec((1, H, 2 * I), lambda i, *sp: (eid(i, *sp), 0, 0)),
            pl.BlockSpec((1, I, H), lambda i, *sp: (eid(i, *sp), 0, 0)),
            pl.BlockSpec((1, 1, R, H),
                         lambda i, *sp: (sp[1][0], eid(i, *sp), 0, 0)),
            pl.BlockSpec((1, 1, I, R),
                         lambda i, *sp: (sp[1][0], eid(i, *sp), 0, 0)),
            pl.BlockSpec((1, 1, R, H),
                         lambda i, *sp: (sp[1][0], eid(i, *sp), 0, 0)),
            pl.BlockSpec((1, 1, I, R),
                         lambda i, *sp: (sp[1][0], eid(i, *sp), 0, 0)),
            pl.BlockSpec((1, 1, R, I),
                         lambda i, *sp: (sp[1][0], eid(i, *sp), 0, 0)),
            pl.BlockSpec((1, 1, H, R),
                         lambda i, *sp: (sp[1][0], eid(i, *sp), 0, 0)),
        ],
        out_specs=pl.BlockSpec((BLK, H), lambda i, *sp: (i, 0)),
    )
    ys = pl.pallas_call(
        functools.partial(_ffn_body, I, E),
        grid_spec=grid_spec,
        out_shape=jax.ShapeDtypeStruct((NPAD, H), jnp.float32),
    )(pad_start, weight_indices.astype(jnp.int32),
      scalings.astype(jnp.float32), xs,
      base_gate_up_weight.astype(jnp.float32),
      base_down_weight.astype(jnp.float32), gate_a, gate_b, up_a, up_b,
      down_a, down_b)

    tw = topk_weights.astype(jnp.float32)
    w0b = jnp.broadcast_to(tw[:, 0:1], (T, 16))
    w1b = jnp.broadcast_to(tw[:, 1:2], (T, 16))
    out = _sc_combine(ys, pos[:, 0], pos[:, 1], w0b, w1b)
    return out.astype(hidden_states.dtype)


# speedup vs baseline: 1.0346x; 1.0346x over previous
import functools

import jax
import jax.numpy as jnp
from jax import lax
from jax.experimental import pallas as pl
from jax.experimental.pallas import tpu as pltpu
from jax.experimental.pallas import tpu_sc as plsc

BLK = 256


def _sc_gather(x, idx, n_rows, chunk):
    _, H = x.shape
    info = plsc.get_sparse_core_info()
    NC, NS = info.num_cores, info.num_subcores
    NW = NC * NS
    per_w = n_rows // NW
    assert n_rows % (NW * chunk) == 0 and chunk % 8 == 0
    nchunks = per_w // chunk
    mesh = plsc.VectorSubcoreMesh(core_axis_name="c", subcore_axis_name="s")

    @functools.partial(
        pl.kernel,
        out_type=jax.ShapeDtypeStruct((n_rows, H), x.dtype),
        mesh=mesh,
        scratch_types=[
            pltpu.VMEM((per_w,), jnp.int32),
            pltpu.VMEM((chunk, H), x.dtype),
            pltpu.VMEM((chunk, H), x.dtype),
            pltpu.SemaphoreType.DMA,
            pltpu.SemaphoreType.DMA,
        ],
    )
    def gather_k(x_hbm, idx_hbm, out_hbm, idx_v, b0, b1, gsem, ssem):
        wid = lax.axis_index("s") * NC + lax.axis_index("c")
        base = wid * per_w
        pltpu.sync_copy(idx_hbm.at[pl.ds(base, per_w)], idx_v)
        bufs = [b0, b1]
        gathers = [None] * nchunks
        stores = [None] * nchunks
        gathers[0] = pltpu.async_copy(
            x_hbm.at[idx_v.at[pl.ds(0, chunk)]], bufs[0], gsem)
        for c in range(nchunks):
            buf = bufs[c % 2]
            gathers[c].wait()
            if c + 1 < nchunks:
                if c >= 1:
                    stores[c - 1].wait()
                gathers[c + 1] = pltpu.async_copy(
                    x_hbm.at[idx_v.at[pl.ds((c + 1) * chunk, chunk)]],
                    bufs[(c + 1) % 2], gsem)
            stores[c] = pltpu.async_copy(
                buf, out_hbm.at[pl.ds(base + c * chunk, chunk)], ssem)
        for c in range(max(0, nchunks - 2), nchunks):
            stores[c].wait()

    return gather_k(x, idx)


def _sc_combine(ys, pos0, pos1, w0, w1):
    T = pos0.shape[0]
    H = ys.shape[1]
    info = plsc.get_sparse_core_info()
    NC, NS = info.num_cores, info.num_subcores
    NW = NC * NS
    per_w = T // NW
    assert T % NW == 0 and per_w % 8 == 0
    mesh = plsc.VectorSubcoreMesh(core_axis_name="c", subcore_axis_name="s")

    @functools.partial(
        pl.kernel,
        out_type=jax.ShapeDtypeStruct((T, H), ys.dtype),
        mesh=mesh,
        scratch_types=[
            pltpu.VMEM((per_w,), jnp.int32),
            pltpu.VMEM((per_w,), jnp.int32),
            pltpu.VMEM((per_w, 16), jnp.float32),
            pltpu.VMEM((per_w, 16), jnp.float32),
            pltpu.VMEM((per_w, H), ys.dtype),
            pltpu.VMEM((per_w, H), ys.dtype),
            pltpu.SemaphoreType.DMA,
        ],
    )
    def combine_k(ys_hbm, p0_hbm, p1_hbm, w0_hbm, w1_hbm, out_hbm, i0, i1,
                  wv0, wv1, b0, b1, sem):
        wid = lax.axis_index("s") * NC + lax.axis_index("c")
        base = wid * per_w
        pltpu.sync_copy(p0_hbm.at[pl.ds(base, per_w)], i0)
        pltpu.sync_copy(p1_hbm.at[pl.ds(base, per_w)], i1)
        pltpu.sync_copy(w0_hbm.at[pl.ds(base, per_w)], wv0)
        pltpu.sync_copy(w1_hbm.at[pl.ds(base, per_w)], wv1)
        cp0 = pltpu.async_copy(ys_hbm.at[i0], b0, sem)
        cp1 = pltpu.async_copy(ys_hbm.at[i1], b1, sem)
        cp0.wait()
        cp1.wait()

        def row(r, carry):
            a0 = wv0[r, :]
            a1 = wv1[r, :]
            for c in range(H // 16):
                s = pl.ds(c * 16, 16)
                b0[r, s] = b0[r, s] * a0 + b1[r, s] * a1
            return carry

        lax.fori_loop(0, per_w, row, 0)
        pltpu.sync_copy(b0, out_hbm.at[pl.ds(base, per_w)])

    return combine_k(ys, pos0, pos1, w0, w1)


def _ffn_body(I, E, ps_ref, xs_ref, wgu_ref, wd_ref, gaT_ref, gbT_ref,
              uaT_ref, ubT_ref, daT_ref, dbT_ref, ys_ref):
    i = pl.program_id(0)

    @pl.when(i * BLK < ps_ref[E])
    def _():
        f32 = jnp.float32
        bf = jnp.bfloat16
        x = xs_ref[...].astype(bf)
        gb_base = jnp.dot(x, wgu_ref[0].astype(bf), preferred_element_type=f32)
        lg = jnp.dot(
            jnp.dot(x, gaT_ref[0].astype(bf),
                    preferred_element_type=f32).astype(bf),
            gbT_ref[0].astype(bf), preferred_element_type=f32)
        lu = jnp.dot(
            jnp.dot(x, uaT_ref[0].astype(bf),
                    preferred_element_type=f32).astype(bf),
            ubT_ref[0].astype(bf), preferred_element_type=f32)
        g = gb_base[:, :I] + lg
        u = gb_base[:, I:] + lu
        a = (g * jax.nn.sigmoid(g) * u).astype(bf)
        d = jnp.dot(a, wd_ref[0].astype(bf), preferred_element_type=f32)
        ld = jnp.dot(
            jnp.dot(a, daT_ref[0].astype(bf),
                    preferred_element_type=f32).astype(bf),
            dbT_ref[0].astype(bf), preferred_element_type=f32)
        ys_ref[...] = d + ld


def kernel(hidden_states, topk_ids, topk_weights, gate_a, gate_b, up_a, up_b,
           down_a, down_b, weight_indices, seq_lens, lora_ranks, scalings,
           base_gate_up_weight, base_down_weight):
    T, H = hidden_states.shape
    K = topk_ids.shape[1]
    E = base_gate_up_weight.shape[0]
    I = base_down_weight.shape[1]
    R = gate_a.shape[2]
    NP = T * K
    NPAD = NP + E * BLK
    NBLK = NPAD // BLK
    NW = 32
    CHUNK = 64

    x = hidden_states.astype(jnp.float32)

    e_flat = topk_ids.reshape(-1).astype(jnp.int32)
    t_flat = jnp.repeat(jnp.arange(T, dtype=jnp.int32), K)
    oh = (e_flat[:, None] == jnp.arange(E, dtype=jnp.int32)[None, :])
    ranks_all = jnp.cumsum(oh.astype(jnp.int32), axis=0)
    counts = ranks_all[-1]
    rank = jnp.take_along_axis(ranks_all, e_flat[:, None], axis=1)[:, 0] - 1
    padded = ((counts + BLK - 1) // BLK) * BLK
    pad_start = jnp.concatenate(
        [jnp.zeros((1,), jnp.int32), jnp.cumsum(padded).astype(jnp.int32)])
    padpos = pad_start[e_flat] + rank
    pos = padpos.reshape(T, K)
    tok_pad = (jnp.arange(NPAD, dtype=jnp.int32) % T).at[padpos].set(t_flat)

    adapter = weight_indices[0]
    sc = scalings[adapter].astype(jnp.float32)
    gaT = jnp.transpose(gate_a[adapter].astype(jnp.float32), (0, 2, 1))
    gbT = jnp.transpose(gate_b[adapter].astype(jnp.float32), (0, 2, 1)) * sc
    uaT = jnp.transpose(up_a[adapter].astype(jnp.float32), (0, 2, 1))
    ubT = jnp.transpose(up_b[adapter].astype(jnp.float32), (0, 2, 1)) * sc
    daT = jnp.transpose(down_a[adapter].astype(jnp.float32), (0, 2, 1))
    dbT = jnp.transpose(down_b[adapter].astype(jnp.float32), (0, 2, 1)) * sc

    xs = _sc_gather(x, tok_pad, NPAD, CHUNK)

    def eid(i, ps):
        st = i * BLK
        e = jnp.int32(0)
        for j in range(1, E):
            e = e + (ps[j] <= st).astype(jnp.int32)
        return e

    grid_spec = pltpu.PrefetchScalarGridSpec(
        num_scalar_prefetch=1,
        grid=(NBLK,),
        in_specs=[
            pl.BlockSpec((BLK, H), lambda i, ps: (i, 0)),
            pl.BlockSpec((1, H, 2 * I), lambda i, ps: (eid(i, ps), 0, 0)),
            pl.BlockSpec((1, I, H), lambda i, ps: (eid(i, ps), 0, 0)),
            pl.BlockSpec((1, H, R), lambda i, ps: (eid(i, ps), 0, 0)),
            pl.BlockSpec((1, R, I), lambda i, ps: (eid(i, ps), 0, 0)),
            pl.BlockSpec((1, H, R), lambda i, ps: (eid(i, ps), 0, 0)),
            pl.BlockSpec((1, R, I), lambda i, ps: (eid(i, ps), 0, 0)),
            pl.BlockSpec((1, I, R), lambda i, ps: (eid(i, ps), 0, 0)),
            pl.BlockSpec((1, R, H), lambda i, ps: (eid(i, ps), 0, 0)),
        ],
        out_specs=pl.BlockSpec((BLK, H), lambda i, ps: (i, 0)),
    )
    ys = pl.pallas_call(
        functools.partial(_ffn_body, I, E),
        grid_spec=grid_spec,
        out_shape=jax.ShapeDtypeStruct((NPAD, H), jnp.float32),
    )(pad_start, xs, base_gate_up_weight.astype(jnp.float32),
      base_down_weight.astype(jnp.float32), gaT, gbT, uaT, ubT, daT, dbT)

    tw = topk_weights.astype(jnp.float32)
    w0b = jnp.broadcast_to(tw[:, 0:1], (T, 16))
    w1b = jnp.broadcast_to(tw[:, 1:2], (T, 16))
    out = _sc_combine(ys, pos[:, 0], pos[:, 1], w0b, w1b)
    return out.astype(hidden_states.dtype)


# speedup vs baseline: 1.0928x; 1.0563x over previous
import functools

import jax
import jax.numpy as jnp
from jax import lax
from jax.experimental import pallas as pl
from jax.experimental.pallas import tpu as pltpu
from jax.experimental.pallas import tpu_sc as plsc

BLK = 256


def _sc_gather(x, idx, n_rows, chunk):
    _, H = x.shape
    info = plsc.get_sparse_core_info()
    NC, NS = info.num_cores, info.num_subcores
    NW = NC * NS
    per_w = n_rows // NW
    assert n_rows % (NW * chunk) == 0 and chunk % 8 == 0
    nchunks = per_w // chunk
    mesh = plsc.VectorSubcoreMesh(core_axis_name="c", subcore_axis_name="s")

    @functools.partial(
        pl.kernel,
        out_type=jax.ShapeDtypeStruct((n_rows, H), x.dtype),
        mesh=mesh,
        scratch_types=[
            pltpu.VMEM((per_w,), jnp.int32),
            pltpu.VMEM((chunk, H), x.dtype),
            pltpu.VMEM((chunk, H), x.dtype),
            pltpu.SemaphoreType.DMA,
            pltpu.SemaphoreType.DMA,
        ],
    )
    def gather_k(x_hbm, idx_hbm, out_hbm, idx_v, b0, b1, gsem, ssem):
        wid = lax.axis_index("s") * NC + lax.axis_index("c")
        base = wid * per_w
        pltpu.sync_copy(idx_hbm.at[pl.ds(base, per_w)], idx_v)
        bufs = [b0, b1]
        gathers = [None] * nchunks
        stores = [None] * nchunks
        gathers[0] = pltpu.async_copy(
            x_hbm.at[idx_v.at[pl.ds(0, chunk)]], bufs[0], gsem)
        for c in range(nchunks):
            buf = bufs[c % 2]
            gathers[c].wait()
            if c + 1 < nchunks:
                if c >= 1:
                    stores[c - 1].wait()
                gathers[c + 1] = pltpu.async_copy(
                    x_hbm.at[idx_v.at[pl.ds((c + 1) * chunk, chunk)]],
                    bufs[(c + 1) % 2], gsem)
            stores[c] = pltpu.async_copy(
                buf, out_hbm.at[pl.ds(base + c * chunk, chunk)], ssem)
        for c in range(max(0, nchunks - 2), nchunks):
            stores[c].wait()

    return gather_k(x, idx)


def _sc_combine(ys, pos0, pos1, w0, w1):
    T = pos0.shape[0]
    H = ys.shape[1]
    info = plsc.get_sparse_core_info()
    NC, NS = info.num_cores, info.num_subcores
    NW = NC * NS
    per_w = T // NW
    assert T % NW == 0 and per_w % 8 == 0
    mesh = plsc.VectorSubcoreMesh(core_axis_name="c", subcore_axis_name="s")

    @functools.partial(
        pl.kernel,
        out_type=jax.ShapeDtypeStruct((T, H), ys.dtype),
        mesh=mesh,
        scratch_types=[
            pltpu.VMEM((per_w,), jnp.int32),
            pltpu.VMEM((per_w,), jnp.int32),
            pltpu.VMEM((per_w, 16), jnp.float32),
            pltpu.VMEM((per_w, 16), jnp.float32),
            pltpu.VMEM((per_w, H), ys.dtype),
            pltpu.VMEM((per_w, H), ys.dtype),
            pltpu.SemaphoreType.DMA,
        ],
    )
    def combine_k(ys_hbm, p0_hbm, p1_hbm, w0_hbm, w1_hbm, out_hbm, i0, i1,
                  wv0, wv1, b0, b1, sem):
        wid = lax.axis_index("s") * NC + lax.axis_index("c")
        base = wid * per_w
        pltpu.sync_copy(p0_hbm.at[pl.ds(base, per_w)], i0)
        pltpu.sync_copy(p1_hbm.at[pl.ds(base, per_w)], i1)
        pltpu.sync_copy(w0_hbm.at[pl.ds(base, per_w)], wv0)
        pltpu.sync_copy(w1_hbm.at[pl.ds(base, per_w)], wv1)
        cp0 = pltpu.async_copy(ys_hbm.at[i0], b0, sem)
        cp1 = pltpu.async_copy(ys_hbm.at[i1], b1, sem)
        cp0.wait()
        cp1.wait()

        def row(r, carry):
            a0 = wv0[r, :]
            a1 = wv1[r, :]
            for c in range(H // 16):
                s = pl.ds(c * 16, 16)
                b0[r, s] = b0[r, s] * a0 + b1[r, s] * a1
            return carry

        lax.fori_loop(0, per_w, row, 0)
        pltpu.sync_copy(b0, out_hbm.at[pl.ds(base, per_w)])

    return combine_k(ys, pos0, pos1, w0, w1)


def _ffn_body(I, E, ps_ref, xs_ref, wgu_ref, wd_ref, gaT_ref, gbT_ref,
              uaT_ref, ubT_ref, daT_ref, dbT_ref, ys_ref):
    i = pl.program_id(0)

    @pl.when(i * BLK < ps_ref[E])
    def _():
        f32 = jnp.float32
        bf = jnp.bfloat16
        x = xs_ref[...].astype(bf)
        gb_base = jnp.dot(x, wgu_ref[0].astype(bf), preferred_element_type=f32)
        lg = jnp.dot(
            jnp.dot(x, gaT_ref[0].astype(bf),
                    preferred_element_type=f32).astype(bf),
            gbT_ref[0].astype(bf), preferred_element_type=f32)
        lu = jnp.dot(
            jnp.dot(x, uaT_ref[0].astype(bf),
                    preferred_element_type=f32).astype(bf),
            ubT_ref[0].astype(bf), preferred_element_type=f32)
        g = gb_base[:, :I] + lg
        u = gb_base[:, I:] + lu
        a = (g * jax.nn.sigmoid(g) * u).astype(bf)
        d = jnp.dot(a, wd_ref[0].astype(bf), preferred_element_type=f32)
        ld = jnp.dot(
            jnp.dot(a, daT_ref[0].astype(bf),
                    preferred_element_type=f32).astype(bf),
            dbT_ref[0].astype(bf), preferred_element_type=f32)
        ys_ref[...] = d + ld


def kernel(hidden_states, topk_ids, topk_weights, gate_a, gate_b, up_a, up_b,
           down_a, down_b, weight_indices, seq_lens, lora_ranks, scalings,
           base_gate_up_weight, base_down_weight):
    T, H = hidden_states.shape
    K = topk_ids.shape[1]
    E = base_gate_up_weight.shape[0]
    I = base_down_weight.shape[1]
    R = gate_a.shape[2]
    NP = T * K
    NPAD = NP + E * BLK
    NBLK = NPAD // BLK
    NW = 32
    CHUNK = 64

    x = hidden_states.astype(jnp.float32)

    e_flat = topk_ids.reshape(-1).astype(jnp.int32)
    t_flat = jnp.repeat(jnp.arange(T, dtype=jnp.int32), K)
    oh = (e_flat[:, None] == jnp.arange(E, dtype=jnp.int32)[None, :]).astype(
        jnp.int32)
    ranks_all = jnp.cumsum(oh, axis=0)
    counts = ranks_all[-1]
    rank = jnp.sum(ranks_all * oh, axis=1) - 1
    padded = ((counts + BLK - 1) // BLK) * BLK
    pad_start = jnp.concatenate(
        [jnp.zeros((1,), jnp.int32), jnp.cumsum(padded).astype(jnp.int32)])
    padpos = jnp.sum(oh * pad_start[None, :E], axis=1) + rank
    pos = padpos.reshape(T, K)
    tok_pad = (jnp.arange(NPAD, dtype=jnp.int32) % T).at[padpos].set(t_flat)

    adapter = weight_indices[0]
    sc = scalings[adapter].astype(jnp.float32)
    gaT = jnp.transpose(gate_a[adapter].astype(jnp.float32), (0, 2, 1))
    gbT = jnp.transpose(gate_b[adapter].astype(jnp.float32), (0, 2, 1)) * sc
    uaT = jnp.transpose(up_a[adapter].astype(jnp.float32), (0, 2, 1))
    ubT = jnp.transpose(up_b[adapter].astype(jnp.float32), (0, 2, 1)) * sc
    daT = jnp.transpose(down_a[adapter].astype(jnp.float32), (0, 2, 1))
    dbT = jnp.transpose(down_b[adapter].astype(jnp.float32), (0, 2, 1)) * sc

    xs = _sc_gather(x, tok_pad, NPAD, CHUNK)

    def eid(i, ps):
        st = i * BLK
        e = jnp.int32(0)
        for j in range(1, E):
            e = e + (ps[j] <= st).astype(jnp.int32)
        return e

    grid_spec = pltpu.PrefetchScalarGridSpec(
        num_scalar_prefetch=1,
        grid=(NBLK,),
        in_specs=[
            pl.BlockSpec((BLK, H), lambda i, ps: (i, 0)),
            pl.BlockSpec((1, H, 2 * I), lambda i, ps: (eid(i, ps), 0, 0)),
            pl.BlockSpec((1, I, H), lambda i, ps: (eid(i, ps), 0, 0)),
            pl.BlockSpec((1, H, R), lambda i, ps: (eid(i, ps), 0, 0)),
            pl.BlockSpec((1, R, I), lambda i, ps: (eid(i, ps), 0, 0)),
            pl.BlockSpec((1, H, R), lambda i, ps: (eid(i, ps), 0, 0)),
            pl.BlockSpec((1, R, I), lambda i, ps: (eid(i, ps), 0, 0)),
            pl.BlockSpec((1, I, R), lambda i, ps: (eid(i, ps), 0, 0)),
            pl.BlockSpec((1, R, H), lambda i, ps: (eid(i, ps), 0, 0)),
        ],
        out_specs=pl.BlockSpec((BLK, H), lambda i, ps: (i, 0)),
    )
    ys = pl.pallas_call(
        functools.partial(_ffn_body, I, E),
        grid_spec=grid_spec,
        out_shape=jax.ShapeDtypeStruct((NPAD, H), jnp.float32),
    )(pad_start, xs, base_gate_up_weight.astype(jnp.float32),
      base_down_weight.astype(jnp.float32), gaT, gbT, uaT, ubT, daT, dbT)

    tw = topk_weights.astype(jnp.float32)
    w0b = jnp.broadcast_to(tw[:, 0:1], (T, 16))
    w1b = jnp.broadcast_to(tw[:, 1:2], (T, 16))
    out = _sc_combine(ys, pos[:, 0], pos[:, 1], w0b, w1b)
    return out.astype(hidden_states.dtype)
